# baseline (speedup 1.0000x reference)
import functools

import jax
import jax.numpy as jnp
from jax.experimental import pallas as pl
from jax.experimental.pallas import tpu as pltpu

_N_BINS = 15

_BOUNDS = jnp.linspace(0.0, 1.0, _N_BINS + 1)
_LOWERS = tuple(float(v) for v in _BOUNDS[:-1].at[0].set(-1.0))
_UPPERS = tuple(float(v) for v in _BOUNDS[1:])


def _ece_tc_kernel(labels_ref, logits_ref, ece_ref, acc_ref, *, nb_total,
                   n_samples):
    nb = pl.program_id(1)

    @pl.when(nb == 0)
    def _init():
        acc_ref[...] = jnp.zeros_like(acc_ref)

    x = logits_ref[0]
    m = jnp.max(x, axis=-1)
    a = jnp.argmax(x, axis=-1)
    s = jnp.sum(jnp.exp(x - m[:, None]), axis=-1)
    conf = 1.0 / s
    labels = labels_ref[0, 0]
    correct = (a == labels).astype(jnp.float32)
    ones = jnp.ones_like(conf)
    zero = jnp.zeros_like(conf)

    for i in range(_N_BINS):
        in_bin = (conf > _LOWERS[i]) & (conf <= _UPPERS[i])
        acc_ref[0, i, :] += jnp.where(in_bin, conf, zero)
        acc_ref[1, i, :] += jnp.where(in_bin, correct, zero)
        acc_ref[2, i, :] += jnp.where(in_bin, ones, zero)

    @pl.when(nb == nb_total - 1)
    def _finish():
        conf_s = jnp.sum(acc_ref[0], axis=-1)
        corr_s = jnp.sum(acc_ref[1], axis=-1)
        cnt = jnp.sum(acc_ref[2], axis=-1)
        ece_in = jnp.abs((conf_s - corr_s) / n_samples)
        ece_t = jnp.sum(jnp.where(cnt > 0, ece_in, 0.0))
        ece_ref[0, 0, :] = jnp.full((128,), ece_t, jnp.float32)


def kernel(logits, labels):
    T, N, C = logits.shape
    R = 512
    while N % R != 0:
        R //= 2
    NB = N // R

    out = pl.pallas_call(
        functools.partial(_ece_tc_kernel, nb_total=NB, n_samples=N),
        grid=(T, NB),
        in_specs=[
            pl.BlockSpec((1, 1, R), lambda t, nb: (nb, 0, 0)),
            pl.BlockSpec((1, R, C), lambda t, nb: (t, nb, 0)),
        ],
        out_specs=pl.BlockSpec((1, 1, 128), lambda t, nb: (t, 0, 0)),
        out_shape=jax.ShapeDtypeStruct((T, 1, 128), jnp.float32),
        scratch_shapes=[pltpu.VMEM((3, 16, R), jnp.float32)],
    )(labels.reshape(NB, 1, R), logits)
    return out[:, 0, 0]


# speedup vs baseline: 4.1533x; 4.1533x over previous
import functools

import jax
import jax.numpy as jnp
import numpy as np
from jax.experimental import pallas as pl
from jax.experimental.pallas import tpu as pltpu

_N_BINS = 15

_BOUNDS = np.arange(_N_BINS + 1, dtype=np.float32) * (
    np.float32(1.0) / np.float32(_N_BINS))
_LOWERS = (-1.0,) + tuple(float(v) for v in _BOUNDS[1:-1]) + (2.0,)
_UPPERS = tuple(float(v) for v in _BOUNDS[1:]) + (3.0,)


def _ece_tc_kernel(labels_ref, logits_ref, ece_ref, acc_ref, *, nb_total,
                   n_samples, c_dim):
    nb = pl.program_id(1)

    @pl.when(nb == 0)
    def _init():
        acc_ref[...] = jnp.zeros_like(acc_ref)

    x = logits_ref[0]
    r_dim = x.shape[0]
    c_pad = (c_dim + 127) // 128 * 128
    if c_pad != c_dim:
        tail = jnp.concatenate(
            [x[:, (c_pad - 128):c_dim],
             jnp.full((r_dim, c_pad - c_dim), -jnp.inf, jnp.float32)], axis=1)
        xc = jnp.concatenate([x[:, :(c_pad - 128)], tail], axis=1)
    else:
        xc = x

    m = jnp.max(xc, axis=-1, keepdims=True)
    e = jnp.exp(xc - m)
    s = jnp.sum(e, axis=-1, keepdims=True)
    conf = 1.0 / s

    iota = jax.lax.broadcasted_iota(jnp.int32, (r_dim, c_pad), 1)
    big = jnp.int32(2**30)
    fidx = jnp.min(jnp.where(xc == m, iota, big), axis=-1, keepdims=True)
    labels = labels_ref[0]
    correct = (fidx == labels).astype(jnp.float32)

    bidx = jax.lax.broadcasted_iota(jnp.int32, (1, 16), 1)
    delta = np.float32(1.0) / np.float32(_N_BINS)
    lo = jnp.where(bidx == 0, -1.0,
                   jnp.where(bidx == 15, 2.0,
                             bidx.astype(jnp.float32) * delta))
    up = jnp.where(bidx == 15, 3.0,
                   (bidx + 1).astype(jnp.float32) * delta)
    in_bin = (conf > lo) & (conf <= up)
    zeros = jnp.zeros((r_dim, 16), jnp.float32)
    acc_ref[0:1, :] += jnp.sum(
        jnp.where(in_bin, jnp.broadcast_to(conf, (r_dim, 16)), zeros),
        axis=0, keepdims=True)
    acc_ref[1:2, :] += jnp.sum(
        jnp.where(in_bin, jnp.broadcast_to(correct, (r_dim, 16)), zeros),
        axis=0, keepdims=True)
    acc_ref[2:3, :] += jnp.sum(in_bin.astype(jnp.float32), axis=0,
                               keepdims=True)

    @pl.when(nb == nb_total - 1)
    def _finish():
        conf_s = acc_ref[0:1, :]
        corr_s = acc_ref[1:2, :]
        cnt = acc_ref[2:3, :]
        ece_in = jnp.abs((conf_s - corr_s) / n_samples)
        ece_t = jnp.sum(jnp.where(cnt > 0, ece_in, 0.0))
        ece_ref[0, 0, :] = jnp.full((128,), ece_t, jnp.float32)


def kernel(logits, labels):
    T, N, C = logits.shape
    R = 512
    while N % R != 0:
        R //= 2
    NB = N // R

    out = pl.pallas_call(
        functools.partial(_ece_tc_kernel, nb_total=NB, n_samples=N, c_dim=C),
        grid=(T, NB),
        in_specs=[
            pl.BlockSpec((1, R, 1), lambda t, nb: (nb, 0, 0)),
            pl.BlockSpec((1, R, C), lambda t, nb: (t, nb, 0)),
        ],
        out_specs=pl.BlockSpec((1, 1, 128), lambda t, nb: (t, 0, 0)),
        out_shape=jax.ShapeDtypeStruct((T, 1, 128), jnp.float32),
        scratch_shapes=[pltpu.VMEM((3, 16), jnp.float32)],
    )(labels.reshape(NB, R, 1), logits)
    return out[:, 0, 0]


# speedup vs baseline: 10.5781x; 2.5469x over previous
import functools

import jax
import jax.numpy as jnp
import numpy as np
from jax.experimental import pallas as pl
from jax.experimental.pallas import tpu as pltpu

_N_BINS = 15
_DELTA = float(np.float32(1.0) / np.float32(_N_BINS))


def _ece_tc_kernel(labels_ref, logits_ref, ece_ref, acc_ref, *, nb_total,
                   n_samples):
    nb = pl.program_id(1)

    @pl.when(nb == 0)
    def _init():
        acc_ref[...] = jnp.zeros_like(acc_ref)

    x = logits_ref[0]
    c_dim, r_dim = x.shape
    m = jnp.max(x, axis=0, keepdims=True)
    e = jnp.exp(x - m)
    s = jnp.sum(e, axis=0, keepdims=True)
    conf = 1.0 / s

    iota = jax.lax.broadcasted_iota(jnp.int32, (c_dim, r_dim), 0)
    big = jnp.int32(2**30)
    fidx = jnp.min(jnp.where(x == m, iota, big), axis=0, keepdims=True)
    labels = labels_ref[0]
    correct = (fidx == labels).astype(jnp.float32)
    ones = jnp.ones((1, r_dim), jnp.float32)

    for i in range(_N_BINS):
        lo = -1.0 if i == 0 else float(np.float32(i) * np.float32(_DELTA))
        up = float(np.float32(i + 1) * np.float32(_DELTA))
        in_bin = (conf > lo) & (conf <= up)
        acc_ref[i:i + 1, :] += jnp.where(in_bin, conf, 0.0)
        acc_ref[16 + i:17 + i, :] += jnp.where(in_bin, correct, 0.0)
        acc_ref[32 + i:33 + i, :] += jnp.where(in_bin, ones, 0.0)

    @pl.when(nb == nb_total - 1)
    def _finish():
        conf_s = jnp.sum(acc_ref[0:16, :], axis=-1)
        corr_s = jnp.sum(acc_ref[16:32, :], axis=-1)
        cnt = jnp.sum(acc_ref[32:48, :], axis=-1)
        ece_in = jnp.abs((conf_s - corr_s) / n_samples)
        ece_t = jnp.sum(jnp.where(cnt > 0, ece_in, 0.0))
        ece_ref[0, 0, :] = jnp.full((128,), ece_t, jnp.float32)


def kernel(logits, labels):
    T, N, C = logits.shape
    R = 512
    while N % R != 0:
        R //= 2
    NB = N // R

    logits_t = jnp.transpose(logits, (0, 2, 1))

    out = pl.pallas_call(
        functools.partial(_ece_tc_kernel, nb_total=NB, n_samples=N),
        grid=(T, NB),
        in_specs=[
            pl.BlockSpec((1, 1, R), lambda t, nb: (nb, 0, 0)),
            pl.BlockSpec((1, C, R), lambda t, nb: (t, 0, nb)),
        ],
        out_specs=pl.BlockSpec((1, 1, 128), lambda t, nb: (t, 0, 0)),
        out_shape=jax.ShapeDtypeStruct((T, 1, 128), jnp.float32),
        scratch_shapes=[pltpu.VMEM((48, R), jnp.float32)],
    )(labels.reshape(NB, 1, R), logits_t)
    return out[:, 0, 0]


# speedup vs baseline: 12.9852x; 1.2276x over previous
import functools

import jax
import jax.numpy as jnp
import numpy as np
from jax.experimental import pallas as pl
from jax.experimental.pallas import tpu as pltpu

_N_BINS = 15
_DELTA = float(np.float32(1.0) / np.float32(_N_BINS))


def _ece_tc_kernel(labels_ref, logits_ref, ece_ref, acc_ref, *, nb_total,
                   n_samples):
    nb = pl.program_id(1)

    @pl.when(nb == 0)
    def _init():
        acc_ref[...] = jnp.zeros_like(acc_ref)

    x = logits_ref[0]
    c_dim, r_dim = x.shape
    m = jnp.max(x, axis=0, keepdims=True)
    e = jnp.exp(x - m)
    s = jnp.sum(e, axis=0, keepdims=True)
    conf = 1.0 / s

    iota = jax.lax.broadcasted_iota(jnp.int32, (c_dim, r_dim), 0)
    big = jnp.int32(2**30)
    fidx = jnp.min(jnp.where(x == m, iota, big), axis=0, keepdims=True)
    labels = labels_ref[0]
    correct = (fidx == labels).astype(jnp.float32)
    ones = jnp.ones((1, r_dim), jnp.float32)

    for i in range(_N_BINS):
        lo = -1.0 if i == 0 else float(np.float32(i) * np.float32(_DELTA))
        up = float(np.float32(i + 1) * np.float32(_DELTA))
        in_bin = (conf > lo) & (conf <= up)
        acc_ref[i:i + 1, :] += jnp.where(in_bin, conf, 0.0)
        acc_ref[16 + i:17 + i, :] += jnp.where(in_bin, correct, 0.0)
        acc_ref[32 + i:33 + i, :] += jnp.where(in_bin, ones, 0.0)

    @pl.when(nb == nb_total - 1)
    def _finish():
        conf_s = jnp.sum(acc_ref[0:16, :], axis=-1)
        corr_s = jnp.sum(acc_ref[16:32, :], axis=-1)
        cnt = jnp.sum(acc_ref[32:48, :], axis=-1)
        ece_in = jnp.abs((conf_s - corr_s) / n_samples)
        ece_t = jnp.sum(jnp.where(cnt > 0, ece_in, 0.0))
        ece_ref[0, 0, :] = jnp.full((128,), ece_t, jnp.float32)


def kernel(logits, labels):
    T, N, C = logits.shape
    R = 1024
    while N % R != 0:
        R //= 2
    NB = N // R

    logits_t = jnp.transpose(logits, (0, 2, 1))

    out = pl.pallas_call(
        functools.partial(_ece_tc_kernel, nb_total=NB, n_samples=N),
        grid=(T, NB),
        in_specs=[
            pl.BlockSpec((1, 1, R), lambda t, nb: (nb, 0, 0)),
            pl.BlockSpec((1, C, R), lambda t, nb: (t, 0, nb)),
        ],
        out_specs=pl.BlockSpec((1, 1, 128), lambda t, nb: (t, 0, 0)),
        out_shape=jax.ShapeDtypeStruct((T, 1, 128), jnp.float32),
        scratch_shapes=[pltpu.VMEM((48, R), jnp.float32)],
    )(labels.reshape(NB, 1, R), logits_t)
    return out[:, 0, 0]


# speedup vs baseline: 14.0865x; 1.0848x over previous
import functools

import jax
import jax.numpy as jnp
import numpy as np
from jax.experimental import pallas as pl
from jax.experimental.pallas import tpu as pltpu

_N_BINS = 15
_DELTA = float(np.float32(1.0) / np.float32(_N_BINS))


def _ece_tc_kernel(labels_ref, logits_ref, ece_ref, acc_ref, *, nb_total,
                   n_samples):
    nb = pl.program_id(1)

    @pl.when(nb == 0)
    def _init():
        acc_ref[...] = jnp.zeros_like(acc_ref)

    x = logits_ref[0]
    c_dim, r_dim = x.shape
    m = jnp.max(x, axis=0, keepdims=True)
    e = jnp.exp(x - m)
    s = jnp.sum(e, axis=0, keepdims=True)
    conf = 1.0 / s

    iota = jax.lax.broadcasted_iota(jnp.int32, (c_dim, r_dim), 0)
    big = jnp.int32(2**30)
    fidx = jnp.min(jnp.where(x == m, iota, big), axis=0, keepdims=True)
    labels = labels_ref[0]
    correct = (fidx == labels).astype(jnp.float32)
    ones = jnp.ones((1, r_dim), jnp.float32)

    for i in range(_N_BINS):
        lo = -1.0 if i == 0 else float(np.float32(i) * np.float32(_DELTA))
        up = float(np.float32(i + 1) * np.float32(_DELTA))
        in_bin = (conf > lo) & (conf <= up)
        acc_ref[i:i + 1, :] += jnp.where(in_bin, conf, 0.0)
        acc_ref[16 + i:17 + i, :] += jnp.where(in_bin, correct, 0.0)
        acc_ref[32 + i:33 + i, :] += jnp.where(in_bin, ones, 0.0)

    @pl.when(nb == nb_total - 1)
    def _finish():
        conf_s = jnp.sum(acc_ref[0:16, :], axis=-1)
        corr_s = jnp.sum(acc_ref[16:32, :], axis=-1)
        cnt = jnp.sum(acc_ref[32:48, :], axis=-1)
        ece_in = jnp.abs((conf_s - corr_s) / n_samples)
        ece_t = jnp.sum(jnp.where(cnt > 0, ece_in, 0.0))
        ece_ref[0, 0, :] = jnp.full((128,), ece_t, jnp.float32)


def kernel(logits, labels):
    T, N, C = logits.shape
    R = 2048
    while N % R != 0:
        R //= 2
    NB = N // R

    logits_t = jnp.transpose(logits, (0, 2, 1))

    out = pl.pallas_call(
        functools.partial(_ece_tc_kernel, nb_total=NB, n_samples=N),
        grid=(T, NB),
        in_specs=[
            pl.BlockSpec((1, 1, R), lambda t, nb: (nb, 0, 0)),
            pl.BlockSpec((1, C, R), lambda t, nb: (t, 0, nb)),
        ],
        out_specs=pl.BlockSpec((1, 1, 128), lambda t, nb: (t, 0, 0)),
        out_shape=jax.ShapeDtypeStruct((T, 1, 128), jnp.float32),
        scratch_shapes=[pltpu.VMEM((48, R), jnp.float32)],
    )(labels.reshape(NB, 1, R), logits_t)
    return out[:, 0, 0]
